# baseline (speedup 1.0000x reference)
import functools

import jax
import jax.numpy as jnp
from jax.experimental import pallas as pl
from jax.experimental.pallas import tpu as pltpu

B, T, D, H, N = 1, 2048, 768, 64, 64


def _moe_body(x_ref, xb_ref, mag_ref, freq_ref, phase_ref, down_ref, rw_ref,
              rb_ref, nw_ref, out_ref, acc_ref, top_ref):
    e = pl.program_id(0)

    @pl.when(e == 0)
    def _():
        scores = jnp.dot(x_ref[:], rw_ref[:].T,
                         preferred_element_type=jnp.float32) + rb_ref[:]
        top_ref[:] = jnp.argmax(scores, axis=-1, keepdims=True).astype(jnp.int32)

    mag = jnp.dot(xb_ref[:], mag_ref[0], preferred_element_type=jnp.float32)
    freq = jnp.dot(x_ref[:], freq_ref[0], preferred_element_type=jnp.float32)
    hidden = jnp.tanh(mag) * jnp.cos(
        jax.nn.softplus(freq) + 0.1 + phase_ref[0, 0])
    o = jnp.dot(hidden.astype(jnp.bfloat16), down_ref[0],
                preferred_element_type=jnp.float32)
    contrib = jnp.where(top_ref[:] == e, o, 0.0)

    @pl.when(e == 0)
    def _():
        acc_ref[:] = contrib

    @pl.when(e > 0)
    def _():
        acc_ref[:] += contrib

    @pl.when(e == N - 1)
    def _():
        a = acc_ref[:]
        var = jnp.mean(a * a, axis=-1, keepdims=True)
        out_ref[:] = a * jax.lax.rsqrt(var + 1e-6) * nw_ref[:]


@functools.partial(jax.jit, static_argnames=())
def kernel(x, bank_mag, bank_freq, bank_phase, bank_down, router_W,
           router_bias, norm_weight):
    xf = x.reshape(T, D)
    xb = xf.astype(jnp.bfloat16)
    magb = bank_mag.astype(jnp.bfloat16)
    downb = bank_down.astype(jnp.bfloat16)
    phase3 = bank_phase.reshape(N, 1, H)
    rb = router_bias.reshape(1, N)
    nw = norm_weight.reshape(1, D)

    out = pl.pallas_call(
        _moe_body,
        grid=(N,),
        in_specs=[
            pl.BlockSpec((T, D), lambda e: (0, 0)),
            pl.BlockSpec((T, D), lambda e: (0, 0)),
            pl.BlockSpec((1, D, H), lambda e: (e, 0, 0)),
            pl.BlockSpec((1, D, H), lambda e: (e, 0, 0)),
            pl.BlockSpec((1, 1, H), lambda e: (e, 0, 0)),
            pl.BlockSpec((1, H, D), lambda e: (e, 0, 0)),
            pl.BlockSpec((N, D), lambda e: (0, 0)),
            pl.BlockSpec((1, N), lambda e: (0, 0)),
            pl.BlockSpec((1, D), lambda e: (0, 0)),
        ],
        out_specs=pl.BlockSpec((T, D), lambda e: (0, 0)),
        out_shape=jax.ShapeDtypeStruct((T, D), jnp.float32),
        scratch_shapes=[
            pltpu.VMEM((T, D), jnp.float32),
            pltpu.VMEM((T, 1), jnp.int32),
        ],
    )(xf, xb, magb, bank_freq, phase3, downb, router_W, rb, nw)
    return out.reshape(B, T, D)


# speedup vs baseline: 3.9242x; 3.9242x over previous
import functools

import jax
import jax.numpy as jnp
from jax import lax
from jax.experimental import pallas as pl
from jax.experimental.pallas import tpu as pltpu
from jax.experimental.pallas import tpu_sc as plsc

B, T, D, H, N = 1, 2048, 768, 64, 64
TM = 64
OFFS_W = 128


def _router_body(x_ref, rw_ref, rb_ref, pos_ref, offs_ref):
    scores = jnp.dot(x_ref[:], rw_ref[:].T,
                     preferred_element_type=jnp.float32) + rb_ref[:]
    eid = jnp.argmax(scores, axis=-1, keepdims=True).astype(jnp.int32)
    onehot = (eid == lax.broadcasted_iota(jnp.int32, (1, N), 1)).astype(jnp.int32)

    counts = jnp.sum(onehot, axis=0, keepdims=True)
    incl = counts
    k = 1
    while k < N:
        shifted = jnp.concatenate(
            [jnp.zeros((1, k), jnp.int32), incl[:, :-k]], axis=1)
        incl = incl + shifted
        k *= 2
    excl = incl - counts

    csum = onehot
    k = 1
    while k < T:
        shifted = jnp.concatenate(
            [jnp.zeros((k, N), jnp.int32), csum[:-k, :]], axis=0)
        csum = csum + shifted
        k *= 2
    pos = jnp.sum(onehot * (excl + csum - 1), axis=1, keepdims=True)
    pos_ref[:] = pos.astype(jnp.int32)
    offs_ref[:] = jnp.concatenate(
        [excl, jnp.full((1, OFFS_W - N), T, jnp.int32)], axis=1)


def _ffn_body(offs_ref, xs_ref, mag_ref, freq_ref, phase_ref, down_ref,
              nw_ref, ys_ref, acc_ref):
    e = pl.program_id(0)

    @pl.when(e == 0)
    def _():
        acc_ref[:] = jnp.zeros_like(acc_ref)

    start = offs_ref[0, e]
    end = offs_ref[0, e + 1]
    a = (start // 8) * 8
    nch = jnp.where(end > start, (end - a + TM - 1) // TM, 0)

    def chunk(i, carry):
        r0 = a + i * TM
        r0c = jnp.minimum(r0, T - TM)
        xc = xs_ref[pl.ds(r0c, TM), :]
        mag = jnp.dot(xc, mag_ref[0], preferred_element_type=jnp.float32)
        freq = jnp.dot(xc, freq_ref[0], preferred_element_type=jnp.float32)
        hidden = jnp.tanh(mag) * jnp.cos(
            jax.nn.softplus(freq) + 0.1 + phase_ref[0, 0])
        o = jnp.dot(hidden, down_ref[0], preferred_element_type=jnp.float32)
        rows = r0c + lax.broadcasted_iota(jnp.int32, (TM, 1), 0)
        m = (rows >= jnp.maximum(start, r0)) & (rows < end)
        acc_ref[pl.ds(r0c, TM), :] += jnp.where(m, o, 0.0)
        return carry

    lax.fori_loop(0, nch, chunk, 0)

    @pl.when(e == N - 1)
    def _():
        y = acc_ref[:]
        var = jnp.mean(y * y, axis=-1, keepdims=True)
        ys_ref[:] = y * lax.rsqrt(var + 1e-6) * nw_ref[:]


def _make_sc_kernels():
    info = plsc.get_sparse_core_info()
    nc, ns = info.num_cores, info.num_subcores
    nw = nc * ns
    bpw = T // nw
    mesh = plsc.VectorSubcoreMesh(core_axis_name="c", subcore_axis_name="s")

    @functools.partial(
        pl.kernel, mesh=mesh,
        out_type=jax.ShapeDtypeStruct((T, D), jnp.float32),
        scratch_types=[
            pltpu.VMEM((bpw,), jnp.int32),
            pltpu.VMEM((bpw, D), jnp.float32),
            pltpu.SemaphoreType.DMA,
        ],
    )
    def dispatch(pos_hbm, x_hbm, xs_hbm, idx_v, rows_v, sem):
        wid = lax.axis_index("s") * nc + lax.axis_index("c")
        base = wid * bpw
        pltpu.sync_copy(pos_hbm.at[pl.ds(base, bpw)], idx_v)
        pltpu.sync_copy(x_hbm.at[pl.ds(base, bpw)], rows_v)
        pltpu.async_copy(rows_v, xs_hbm.at[idx_v], sem).wait()

    @functools.partial(
        pl.kernel, mesh=mesh,
        out_type=jax.ShapeDtypeStruct((T, D), jnp.float32),
        scratch_types=[
            pltpu.VMEM((bpw,), jnp.int32),
            pltpu.VMEM((bpw, D), jnp.float32),
            pltpu.SemaphoreType.DMA,
        ],
    )
    def combine(pos_hbm, ys_hbm, out_hbm, idx_v, rows_v, sem):
        wid = lax.axis_index("s") * nc + lax.axis_index("c")
        base = wid * bpw
        pltpu.sync_copy(pos_hbm.at[pl.ds(base, bpw)], idx_v)
        pltpu.async_copy(ys_hbm.at[idx_v], rows_v, sem).wait()
        pltpu.sync_copy(rows_v, out_hbm.at[pl.ds(base, bpw)])

    return dispatch, combine


def _router(xf, router_W, rb):
    return pl.pallas_call(
        _router_body,
        in_specs=[
            pl.BlockSpec((T, D), lambda: (0, 0)),
            pl.BlockSpec((N, D), lambda: (0, 0)),
            pl.BlockSpec((1, N), lambda: (0, 0)),
        ],
        out_specs=[
            pl.BlockSpec((T, 1), lambda: (0, 0)),
            pl.BlockSpec((1, OFFS_W), lambda: (0, 0)),
        ],
        out_shape=[
            jax.ShapeDtypeStruct((T, 1), jnp.int32),
            jax.ShapeDtypeStruct((1, OFFS_W), jnp.int32),
        ],
    )(xf, router_W, rb)


def _ffn(offs, xs, bank_mag, bank_freq, phase3, bank_down, nw):
    return pl.pallas_call(
        _ffn_body,
        grid=(N,),
        in_specs=[
            pl.BlockSpec(memory_space=pltpu.SMEM),
            pl.BlockSpec((T, D), lambda e: (0, 0)),
            pl.BlockSpec((1, D, H), lambda e: (e, 0, 0)),
            pl.BlockSpec((1, D, H), lambda e: (e, 0, 0)),
            pl.BlockSpec((1, 1, H), lambda e: (e, 0, 0)),
            pl.BlockSpec((1, H, D), lambda e: (e, 0, 0)),
            pl.BlockSpec((1, D), lambda e: (0, 0)),
        ],
        out_specs=pl.BlockSpec((T, D), lambda e: (0, 0)),
        out_shape=jax.ShapeDtypeStruct((T, D), jnp.float32),
        scratch_shapes=[pltpu.VMEM((T, D), jnp.float32)],
    )(offs, xs, bank_mag, bank_freq, phase3, bank_down, nw)


@jax.jit
def kernel(x, bank_mag, bank_freq, bank_phase, bank_down, router_W,
           router_bias, norm_weight):
    xf = x.reshape(T, D)
    phase3 = bank_phase.reshape(N, 1, H)
    rb = router_bias.reshape(1, N)
    nw = norm_weight.reshape(1, D)

    pos2d, offs = _router(xf, router_W, rb)
    pos = pos2d.reshape(T)

    dispatch, combine = _make_sc_kernels()
    xs = dispatch(pos, xf)
    ys = _ffn(offs, xs, bank_mag, bank_freq, phase3, bank_down, nw)
    out = combine(pos, ys)
    return out.reshape(B, T, D)
